# Initial kernel scaffold; baseline (speedup 1.0000x reference)
import jax
import jax.numpy as jnp
from jax.experimental import pallas as pl


def kernel(x, edge_index, batch, W1l, b1, W1r, g1, be1, W2l, b2, W2r, g2, be2, W3l, b3, W3r, g3, be3, Wf1, bf1, Wf2, bf2):
    raise NotImplementedError("write your pallas kernel here")


# speedup vs baseline: 8.6856x; 8.6856x over previous
import functools

import jax
import jax.numpy as jnp
from jax import lax
from jax.experimental import pallas as pl
from jax.experimental.pallas import tpu as pltpu
from jax.experimental.pallas import tpu_sc as plsc

N, E, F, G, OUT = 10000, 320000, 128, 64, 10
NC, NS = 2, 16
NW = NC * NS
EW = E // NW
K = 80
NCHUNK = EW // K
SLC = 632
EPS = 1e-5


def _sc_agg(d, with_deg):
    mesh = plsc.VectorSubcoreMesh(core_axis_name="c", subcore_axis_name="s")
    outs = [jax.ShapeDtypeStruct((N, d), jnp.float32),
            jax.ShapeDtypeStruct((N, d), jnp.float32)]
    scratch = [
        pltpu.VMEM((NCHUNK, K), jnp.int32),
        pltpu.VMEM((NCHUNK, K), jnp.int32),
        pltpu.VMEM((K, d), jnp.float32),
        pltpu.VMEM_SHARED((N, d), jnp.float32),
        pltpu.SemaphoreType.DMA,
    ]
    if with_deg:
        outs += [jax.ShapeDtypeStruct((N, 16), jnp.float32),
                 jax.ShapeDtypeStruct((N, 16), jnp.float32)]
        scratch += [
            pltpu.VMEM((K, 16), jnp.float32),
            pltpu.VMEM_SHARED((N, 16), jnp.float32),
        ]

    def body(*refs):
        if with_deg:
            (vh, sh, dh, zh, ones_h, out0, out1, dg0, dg1,
             srcv, dstv, rows, acc, sem, onesv, dacc) = refs
        else:
            (vh, sh, dh, zh, out0, out1,
             srcv, dstv, rows, acc, sem) = refs
        c = lax.axis_index("c")
        s = lax.axis_index("s")
        wid = s * NC + c
        off = pl.multiple_of(jnp.minimum(s * SLC, N - SLC), 8)
        sl = pl.ds(off, SLC)
        pltpu.sync_copy(zh, acc.at[sl])
        if with_deg:
            pltpu.sync_copy(zh, dacc.at[sl])
            pltpu.sync_copy(ones_h, onesv)
        pltpu.sync_copy(sh.at[wid], srcv)
        pltpu.sync_copy(dh.at[wid], dstv)
        plsc.subcore_barrier()

        def chunk(j, carry):
            pltpu.async_copy(vh.at[srcv.at[j]], rows, sem).wait()
            pltpu.sync_copy(rows, acc.at[dstv.at[j]], add=True)
            if with_deg:
                pltpu.sync_copy(onesv, dacc.at[dstv.at[j]], add=True)
            return carry

        lax.fori_loop(0, NCHUNK, chunk, 0)
        plsc.subcore_barrier()

        @pl.when(c == 0)
        def _():
            pltpu.sync_copy(acc.at[sl], out0.at[sl])
            if with_deg:
                pltpu.sync_copy(dacc.at[sl], dg0.at[sl])

        @pl.when(c == 1)
        def _():
            pltpu.sync_copy(acc.at[sl], out1.at[sl])
            if with_deg:
                pltpu.sync_copy(dacc.at[sl], dg1.at[sl])

    return pl.kernel(
        body, out_type=outs, scratch_types=scratch, mesh=mesh,
        compiler_params=pltpu.CompilerParams(use_tc_tiling_on_sc=False))


def _dotT(a, w):
    return lax.dot_general(a, w, (((1,), (1,)), ((), ())),
                           preferred_element_type=jnp.float32)


def _pre1(x, W1l, W1r):
    R = 1000
    NB = N // R

    def body(x_ref, wl_ref, wr_ref, p_ref, r_ref):
        xb = x_ref[...]
        p_ref[...] = _dotT(xb, wl_ref[...])
        r_ref[...] = _dotT(xb, wr_ref[...])

    return pl.pallas_call(
        body,
        grid=(NB,),
        in_specs=[pl.BlockSpec((R, F), lambda i: (i, 0)),
                  pl.BlockSpec((16, F), lambda i: (0, 0)),
                  pl.BlockSpec((16, F), lambda i: (0, 0))],
        out_specs=[pl.BlockSpec((R, 16), lambda i: (i, 0)),
                   pl.BlockSpec((R, 16), lambda i: (i, 0))],
        out_shape=[jax.ShapeDtypeStruct((N, 16), jnp.float32),
                   jax.ShapeDtypeStruct((N, 16), jnp.float32)],
    )(x, W1l, W1r)


def _combine(a0, a1, d0, d1, root, Wl, b):
    do = root.shape[1]
    R = 1000
    NB = N // R
    have_w = Wl is not None

    def body(*refs):
        if have_w:
            (a0_ref, a1_ref, d0_ref, d1_ref, r_ref, w_ref, b_ref,
             h_ref, st_ref, accs) = refs
        else:
            (a0_ref, a1_ref, d0_ref, d1_ref, r_ref, b_ref,
             h_ref, st_ref, accs) = refs
        i = pl.program_id(0)

        @pl.when(i == 0)
        def _():
            accs[...] = jnp.zeros_like(accs)

        agg = a0_ref[...] + a1_ref[...]
        deg = jnp.maximum(d0_ref[:, :1] + d1_ref[:, :1], 1.0)
        mean = agg / deg
        if have_w:
            mean = _dotT(mean, w_ref[...])
        h = mean + b_ref[...] + r_ref[...]
        h_ref[...] = h
        accs[0:1, :] += jnp.sum(h, axis=0, keepdims=True)
        accs[1:2, :] += jnp.sum(h * h, axis=0, keepdims=True)

        @pl.when(i == NB - 1)
        def _():
            st_ref[...] = accs[...]

    da = a0.shape[1]
    in_specs = [pl.BlockSpec((R, da), lambda i: (i, 0)),
                pl.BlockSpec((R, da), lambda i: (i, 0)),
                pl.BlockSpec((R, 16), lambda i: (i, 0)),
                pl.BlockSpec((R, 16), lambda i: (i, 0)),
                pl.BlockSpec((R, do), lambda i: (i, 0))]
    args = [a0, a1, d0, d1, root]
    if have_w:
        in_specs.append(pl.BlockSpec(Wl.shape, lambda i: (0, 0)))
        args.append(Wl)
    in_specs.append(pl.BlockSpec((1, do), lambda i: (0, 0)))
    args.append(b)
    return pl.pallas_call(
        body,
        grid=(NB,),
        in_specs=in_specs,
        out_specs=[pl.BlockSpec((R, do), lambda i: (i, 0)),
                   pl.BlockSpec((8, do), lambda i: (0, 0))],
        out_shape=[jax.ShapeDtypeStruct((N, do), jnp.float32),
                   jax.ShapeDtypeStruct((8, do), jnp.float32)],
        scratch_shapes=[pltpu.VMEM((8, do), jnp.float32)],
    )(*args)


def _norm(hpre, st, g, be, W_next):
    do = hpre.shape[1]
    R = 1000
    NB = N // R
    have_w = W_next is not None

    def body(*refs):
        if have_w:
            h_ref, st_ref, g_ref, be_ref, w_ref, o_ref, p_ref = refs
        else:
            h_ref, st_ref, g_ref, be_ref, o_ref = refs
        mu = st_ref[0:1, :] / N
        var = st_ref[1:2, :] / N - mu * mu
        scale = g_ref[...] * lax.rsqrt(var + EPS)
        h = jnp.maximum((h_ref[...] - mu) * scale + be_ref[...], 0.0)
        o_ref[...] = h
        if have_w:
            p_ref[...] = _dotT(h, w_ref[...])

    in_specs = [pl.BlockSpec((R, do), lambda i: (i, 0)),
                pl.BlockSpec((8, do), lambda i: (0, 0)),
                pl.BlockSpec((1, do), lambda i: (0, 0)),
                pl.BlockSpec((1, do), lambda i: (0, 0))]
    args = [hpre, st, g, be]
    out_specs = [pl.BlockSpec((R, do), lambda i: (i, 0))]
    out_shape = [jax.ShapeDtypeStruct((N, do), jnp.float32)]
    if have_w:
        dn = W_next.shape[0]
        in_specs.append(pl.BlockSpec(W_next.shape, lambda i: (0, 0)))
        args.append(W_next)
        out_specs.append(pl.BlockSpec((R, dn), lambda i: (i, 0)))
        out_shape.append(jax.ShapeDtypeStruct((N, dn), jnp.float32))
    res = pl.pallas_call(
        body,
        grid=(NB,),
        in_specs=in_specs,
        out_specs=out_specs,
        out_shape=out_shape,
    )(*args)
    return res if have_w else res[0]


def _combine3(a0, a1, d0, d1, h2, W3l, W3r, b):
    R = 1000
    NB = N // R

    def body(a0_ref, a1_ref, d0_ref, d1_ref, h2_ref, wl_ref, wr_ref, b_ref,
             h_ref, st_ref, accs):
        i = pl.program_id(0)

        @pl.when(i == 0)
        def _():
            accs[...] = jnp.zeros_like(accs)

        deg = jnp.maximum(d0_ref[:, :1] + d1_ref[:, :1], 1.0)
        mean = (a0_ref[...] + a1_ref[...]) / deg
        h = _dotT(mean, wl_ref[...]) + b_ref[...] + _dotT(h2_ref[...], wr_ref[...])
        h_ref[...] = h
        accs[0:1, :] += jnp.sum(h, axis=0, keepdims=True)
        accs[1:2, :] += jnp.sum(h * h, axis=0, keepdims=True)

        @pl.when(i == NB - 1)
        def _():
            st_ref[...] = accs[...]

    return pl.pallas_call(
        body,
        grid=(NB,),
        in_specs=[pl.BlockSpec((R, 64), lambda i: (i, 0)),
                  pl.BlockSpec((R, 64), lambda i: (i, 0)),
                  pl.BlockSpec((R, 16), lambda i: (i, 0)),
                  pl.BlockSpec((R, 16), lambda i: (i, 0)),
                  pl.BlockSpec((R, 64), lambda i: (i, 0)),
                  pl.BlockSpec((512, 64), lambda i: (0, 0)),
                  pl.BlockSpec((512, 64), lambda i: (0, 0)),
                  pl.BlockSpec((1, 512), lambda i: (0, 0))],
        out_specs=[pl.BlockSpec((R, 512), lambda i: (i, 0)),
                   pl.BlockSpec((8, 512), lambda i: (0, 0))],
        out_shape=[jax.ShapeDtypeStruct((N, 512), jnp.float32),
                   jax.ShapeDtypeStruct((8, 512), jnp.float32)],
        scratch_shapes=[pltpu.VMEM((8, 512), jnp.float32)],
    )(a0, a1, d0, d1, h2, W3l, W3r, b)


def _final(h3pre, batchf, st, g, be, Wf1, bf1, Wf2p, bf2p):
    R = 400
    NB = N // R

    def body(h_ref, b_ref, st_ref, g_ref, be_ref, w1_ref, b1_ref,
             w2_ref, b2_ref, o_ref, acc):
        i = pl.program_id(0)

        @pl.when(i == 0)
        def _():
            acc[...] = jnp.zeros_like(acc)

        mu = st_ref[0:1, :] / N
        var = st_ref[1:2, :] / N - mu * mu
        scale = g_ref[...] * lax.rsqrt(var + EPS)
        h = jnp.maximum((h_ref[...] - mu) * scale + be_ref[...], 0.0)
        bb = b_ref[...]
        cmin = jnp.min(bb).astype(jnp.int32)
        cmax = jnp.max(bb).astype(jnp.int32)

        def upd(c, carry):
            m = (bb == c.astype(jnp.float32))
            contrib = jnp.max(jnp.where(m, h, 0.0), axis=0, keepdims=True)
            row = pl.ds(c, 1)
            acc[row, :] = jnp.maximum(acc[row, :], contrib)
            return carry

        lax.fori_loop(cmin, cmax + 1, upd, 0)

        @pl.when(i == NB - 1)
        def _():
            pooled = acc[...]
            t = jnp.maximum(_dotT(pooled, w1_ref[...]) + b1_ref[...], 0.0)
            o_ref[...] = _dotT(t, w2_ref[...]) + b2_ref[...]

    return pl.pallas_call(
        body,
        grid=(NB,),
        in_specs=[pl.BlockSpec((R, 512), lambda i: (i, 0)),
                  pl.BlockSpec((R, 1), lambda i: (i, 0)),
                  pl.BlockSpec((8, 512), lambda i: (0, 0)),
                  pl.BlockSpec((1, 512), lambda i: (0, 0)),
                  pl.BlockSpec((1, 512), lambda i: (0, 0)),
                  pl.BlockSpec((256, 512), lambda i: (0, 0)),
                  pl.BlockSpec((1, 256), lambda i: (0, 0)),
                  pl.BlockSpec((16, 256), lambda i: (0, 0)),
                  pl.BlockSpec((1, 16), lambda i: (0, 0))],
        out_specs=pl.BlockSpec((G, 16), lambda i: (0, 0)),
        out_shape=jax.ShapeDtypeStruct((G, 16), jnp.float32),
        scratch_shapes=[pltpu.VMEM((G, 512), jnp.float32)],
    )(h3pre, batchf, st, g, be, Wf1, bf1, Wf2p, bf2p)


def kernel(x, edge_index, batch, W1l, b1, W1r, g1, be1, W2l, b2, W2r, g2, be2,
           W3l, b3, W3r, g3, be3, Wf1, bf1, Wf2, bf2):
    src = edge_index[0].reshape(NW, NCHUNK, K)
    dst = edge_index[1].reshape(NW, NCHUNK, K)
    z16 = jnp.zeros((SLC, 16), jnp.float32)
    z64 = jnp.zeros((SLC, 64), jnp.float32)
    onesK = jnp.ones((K, 16), jnp.float32)

    p1, r1 = _pre1(x, W1l, W1r)
    a10, a11, dg0, dg1 = _sc_agg(16, True)(p1, src, dst, z16, onesK)
    h1pre, st1 = _combine(a10, a11, dg0, dg1, r1, None, b1.reshape(1, 16))
    h1, r2 = _norm(h1pre, st1, g1.reshape(1, 16), be1.reshape(1, 16), W2r)

    a20, a21 = _sc_agg(16, False)(h1, src, dst, z16)
    h2pre, st2 = _combine(a20, a21, dg0, dg1, r2, W2l, b2.reshape(1, 64))
    h2 = _norm(h2pre, st2, g2.reshape(1, 64), be2.reshape(1, 64), None)

    a30, a31 = _sc_agg(64, False)(h2, src, dst, z64)
    h3pre, st3 = _combine3(a30, a31, dg0, dg1, h2, W3l, W3r,
                           b3.reshape(1, 512))

    Wf2p = jnp.zeros((16, 256), jnp.float32).at[:OUT].set(Wf2)
    bf2p = jnp.zeros((1, 16), jnp.float32).at[0, :OUT].set(bf2)
    out = _final(h3pre, batch.astype(jnp.float32).reshape(N, 1), st3,
                 g3.reshape(1, 512), be3.reshape(1, 512), Wf1,
                 bf1.reshape(1, 256), Wf2p, bf2p)
    return out[:, :OUT]


# speedup vs baseline: 13.5607x; 1.5613x over previous
import functools

import jax
import jax.numpy as jnp
from jax import lax
from jax.experimental import pallas as pl
from jax.experimental.pallas import tpu as pltpu
from jax.experimental.pallas import tpu_sc as plsc

N, E, F, G, OUT = 10000, 320000, 128, 64, 10
NC, NS = 2, 16
NW = NC * NS
EW = E // NW
K = 125
NCHUNK = EW // K
SLC = 632
EPS = 1e-5


def _sc_agg(d, with_deg):
    mesh = plsc.VectorSubcoreMesh(core_axis_name="c", subcore_axis_name="s")
    outs = [jax.ShapeDtypeStruct((N, d), jnp.float32),
            jax.ShapeDtypeStruct((N, d), jnp.float32)]
    scratch = [
        pltpu.VMEM((NCHUNK, K), jnp.int32),
        pltpu.VMEM((NCHUNK, K), jnp.int32),
        pltpu.VMEM((K, d), jnp.float32),
        pltpu.VMEM((K, d), jnp.float32),
        pltpu.VMEM_SHARED((N, d), jnp.float32),
        pltpu.SemaphoreType.DMA,
        pltpu.SemaphoreType.DMA,
    ]
    if with_deg:
        outs += [jax.ShapeDtypeStruct((N, 16), jnp.float32),
                 jax.ShapeDtypeStruct((N, 16), jnp.float32)]
        scratch += [
            pltpu.VMEM((K, 16), jnp.float32),
            pltpu.VMEM_SHARED((N, 16), jnp.float32),
        ]

    def body(*refs):
        if with_deg:
            (vh, sh, dh, zh, ones_h, out0, out1, dg0, dg1,
             srcv, dstv, rows0, rows1, acc, sem0, sem1, onesv, dacc) = refs
        else:
            (vh, sh, dh, zh, out0, out1,
             srcv, dstv, rows0, rows1, acc, sem0, sem1) = refs
        c = lax.axis_index("c")
        s = lax.axis_index("s")
        wid = s * NC + c
        off = pl.multiple_of(jnp.minimum(s * SLC, N - SLC), 8)
        sl = pl.ds(off, SLC)
        pltpu.sync_copy(zh, acc.at[sl])
        if with_deg:
            pltpu.sync_copy(zh, dacc.at[sl])
            pltpu.sync_copy(ones_h, onesv)
        pltpu.sync_copy(sh.at[wid], srcv)
        pltpu.sync_copy(dh.at[wid], dstv)
        pltpu.async_copy(vh.at[srcv.at[0]], rows0, sem0)
        plsc.subcore_barrier()

        def scat(rows, j):
            pltpu.sync_copy(rows, acc.at[dstv.at[j]], add=True)
            if with_deg:
                pltpu.sync_copy(onesv, dacc.at[dstv.at[j]], add=True)

        def pair(i, carry):
            j = i * 2
            pltpu.async_copy(vh.at[srcv.at[j + 1]], rows1, sem1)
            pltpu.make_async_copy(vh.at[srcv.at[j]], rows0, sem0).wait()
            scat(rows0, j)

            @pl.when(j + 2 < NCHUNK)
            def _():
                pltpu.async_copy(vh.at[srcv.at[j + 2]], rows0, sem0)

            pltpu.make_async_copy(vh.at[srcv.at[j + 1]], rows1, sem1).wait()
            scat(rows1, j + 1)
            return carry

        lax.fori_loop(0, NCHUNK // 2, pair, 0)
        plsc.subcore_barrier()

        @pl.when(c == 0)
        def _():
            pltpu.sync_copy(acc.at[sl], out0.at[sl])
            if with_deg:
                pltpu.sync_copy(dacc.at[sl], dg0.at[sl])

        @pl.when(c == 1)
        def _():
            pltpu.sync_copy(acc.at[sl], out1.at[sl])
            if with_deg:
                pltpu.sync_copy(dacc.at[sl], dg1.at[sl])

    return pl.kernel(
        body, out_type=outs, scratch_types=scratch, mesh=mesh,
        compiler_params=pltpu.CompilerParams(use_tc_tiling_on_sc=False))


def _dotT(a, w):
    return lax.dot_general(a, w, (((1,), (1,)), ((), ())),
                           preferred_element_type=jnp.float32)


def _pre1(x, W1l, W1r):
    R = 1000
    NB = N // R

    def body(x_ref, wl_ref, wr_ref, p_ref, r_ref):
        xb = x_ref[...]
        p_ref[...] = _dotT(xb, wl_ref[...])
        r_ref[...] = _dotT(xb, wr_ref[...])

    return pl.pallas_call(
        body,
        grid=(NB,),
        in_specs=[pl.BlockSpec((R, F), lambda i: (i, 0)),
                  pl.BlockSpec((16, F), lambda i: (0, 0)),
                  pl.BlockSpec((16, F), lambda i: (0, 0))],
        out_specs=[pl.BlockSpec((R, 16), lambda i: (i, 0)),
                   pl.BlockSpec((R, 16), lambda i: (i, 0))],
        out_shape=[jax.ShapeDtypeStruct((N, 16), jnp.float32),
                   jax.ShapeDtypeStruct((N, 16), jnp.float32)],
    )(x, W1l, W1r)


def _combine(a0, a1, d0, d1, root, Wl, b):
    do = root.shape[1]
    R = 1000
    NB = N // R
    have_w = Wl is not None

    def body(*refs):
        if have_w:
            (a0_ref, a1_ref, d0_ref, d1_ref, r_ref, w_ref, b_ref,
             h_ref, st_ref, accs) = refs
        else:
            (a0_ref, a1_ref, d0_ref, d1_ref, r_ref, b_ref,
             h_ref, st_ref, accs) = refs
        i = pl.program_id(0)

        @pl.when(i == 0)
        def _():
            accs[...] = jnp.zeros_like(accs)

        agg = a0_ref[...] + a1_ref[...]
        deg = jnp.maximum(d0_ref[:, :1] + d1_ref[:, :1], 1.0)
        mean = agg / deg
        if have_w:
            mean = _dotT(mean, w_ref[...])
        h = mean + b_ref[...] + r_ref[...]
        h_ref[...] = h
        accs[0:1, :] += jnp.sum(h, axis=0, keepdims=True)
        accs[1:2, :] += jnp.sum(h * h, axis=0, keepdims=True)

        @pl.when(i == NB - 1)
        def _():
            st_ref[...] = accs[...]

    da = a0.shape[1]
    in_specs = [pl.BlockSpec((R, da), lambda i: (i, 0)),
                pl.BlockSpec((R, da), lambda i: (i, 0)),
                pl.BlockSpec((R, 16), lambda i: (i, 0)),
                pl.BlockSpec((R, 16), lambda i: (i, 0)),
                pl.BlockSpec((R, do), lambda i: (i, 0))]
    args = [a0, a1, d0, d1, root]
    if have_w:
        in_specs.append(pl.BlockSpec(Wl.shape, lambda i: (0, 0)))
        args.append(Wl)
    in_specs.append(pl.BlockSpec((1, do), lambda i: (0, 0)))
    args.append(b)
    return pl.pallas_call(
        body,
        grid=(NB,),
        in_specs=in_specs,
        out_specs=[pl.BlockSpec((R, do), lambda i: (i, 0)),
                   pl.BlockSpec((8, do), lambda i: (0, 0))],
        out_shape=[jax.ShapeDtypeStruct((N, do), jnp.float32),
                   jax.ShapeDtypeStruct((8, do), jnp.float32)],
        scratch_shapes=[pltpu.VMEM((8, do), jnp.float32)],
    )(*args)


def _norm(hpre, st, g, be, W_next):
    do = hpre.shape[1]
    R = 1000
    NB = N // R
    have_w = W_next is not None

    def body(*refs):
        if have_w:
            h_ref, st_ref, g_ref, be_ref, w_ref, o_ref, p_ref = refs
        else:
            h_ref, st_ref, g_ref, be_ref, o_ref = refs
        mu = st_ref[0:1, :] / N
        var = st_ref[1:2, :] / N - mu * mu
        scale = g_ref[...] * lax.rsqrt(var + EPS)
        h = jnp.maximum((h_ref[...] - mu) * scale + be_ref[...], 0.0)
        o_ref[...] = h
        if have_w:
            p_ref[...] = _dotT(h, w_ref[...])

    in_specs = [pl.BlockSpec((R, do), lambda i: (i, 0)),
                pl.BlockSpec((8, do), lambda i: (0, 0)),
                pl.BlockSpec((1, do), lambda i: (0, 0)),
                pl.BlockSpec((1, do), lambda i: (0, 0))]
    args = [hpre, st, g, be]
    out_specs = [pl.BlockSpec((R, do), lambda i: (i, 0))]
    out_shape = [jax.ShapeDtypeStruct((N, do), jnp.float32)]
    if have_w:
        dn = W_next.shape[0]
        in_specs.append(pl.BlockSpec(W_next.shape, lambda i: (0, 0)))
        args.append(W_next)
        out_specs.append(pl.BlockSpec((R, dn), lambda i: (i, 0)))
        out_shape.append(jax.ShapeDtypeStruct((N, dn), jnp.float32))
    res = pl.pallas_call(
        body,
        grid=(NB,),
        in_specs=in_specs,
        out_specs=out_specs,
        out_shape=out_shape,
    )(*args)
    return res if have_w else res[0]


def _combine3(a0, a1, d0, d1, h2, W3l, W3r, b):
    R = 1000
    NB = N // R

    def body(a0_ref, a1_ref, d0_ref, d1_ref, h2_ref, wl_ref, wr_ref, b_ref,
             h_ref, st_ref, accs):
        i = pl.program_id(0)

        @pl.when(i == 0)
        def _():
            accs[...] = jnp.zeros_like(accs)

        deg = jnp.maximum(d0_ref[:, :1] + d1_ref[:, :1], 1.0)
        mean = (a0_ref[...] + a1_ref[...]) / deg
        h = _dotT(mean, wl_ref[...]) + b_ref[...] + _dotT(h2_ref[...], wr_ref[...])
        h_ref[...] = h
        accs[0:1, :] += jnp.sum(h, axis=0, keepdims=True)
        accs[1:2, :] += jnp.sum(h * h, axis=0, keepdims=True)

        @pl.when(i == NB - 1)
        def _():
            st_ref[...] = accs[...]

    return pl.pallas_call(
        body,
        grid=(NB,),
        in_specs=[pl.BlockSpec((R, 64), lambda i: (i, 0)),
                  pl.BlockSpec((R, 64), lambda i: (i, 0)),
                  pl.BlockSpec((R, 16), lambda i: (i, 0)),
                  pl.BlockSpec((R, 16), lambda i: (i, 0)),
                  pl.BlockSpec((R, 64), lambda i: (i, 0)),
                  pl.BlockSpec((512, 64), lambda i: (0, 0)),
                  pl.BlockSpec((512, 64), lambda i: (0, 0)),
                  pl.BlockSpec((1, 512), lambda i: (0, 0))],
        out_specs=[pl.BlockSpec((R, 512), lambda i: (i, 0)),
                   pl.BlockSpec((8, 512), lambda i: (0, 0))],
        out_shape=[jax.ShapeDtypeStruct((N, 512), jnp.float32),
                   jax.ShapeDtypeStruct((8, 512), jnp.float32)],
        scratch_shapes=[pltpu.VMEM((8, 512), jnp.float32)],
    )(a0, a1, d0, d1, h2, W3l, W3r, b)


def _final(h3pre, batchf, st, g, be, Wf1, bf1, Wf2p, bf2p):
    R = 400
    NB = N // R

    def body(h_ref, b_ref, st_ref, g_ref, be_ref, w1_ref, b1_ref,
             w2_ref, b2_ref, o_ref, acc):
        i = pl.program_id(0)

        @pl.when(i == 0)
        def _():
            acc[...] = jnp.zeros_like(acc)

        mu = st_ref[0:1, :] / N
        var = st_ref[1:2, :] / N - mu * mu
        scale = g_ref[...] * lax.rsqrt(var + EPS)
        h = jnp.maximum((h_ref[...] - mu) * scale + be_ref[...], 0.0)
        bb = b_ref[...]
        cmin = jnp.min(bb).astype(jnp.int32)
        cmax = jnp.max(bb).astype(jnp.int32)

        def upd(c, carry):
            m = (bb == c.astype(jnp.float32))
            contrib = jnp.max(jnp.where(m, h, 0.0), axis=0, keepdims=True)
            row = pl.ds(c, 1)
            acc[row, :] = jnp.maximum(acc[row, :], contrib)
            return carry

        lax.fori_loop(cmin, cmax + 1, upd, 0)

        @pl.when(i == NB - 1)
        def _():
            pooled = acc[...]
            t = jnp.maximum(_dotT(pooled, w1_ref[...]) + b1_ref[...], 0.0)
            o_ref[...] = _dotT(t, w2_ref[...]) + b2_ref[...]

    return pl.pallas_call(
        body,
        grid=(NB,),
        in_specs=[pl.BlockSpec((R, 512), lambda i: (i, 0)),
                  pl.BlockSpec((R, 1), lambda i: (i, 0)),
                  pl.BlockSpec((8, 512), lambda i: (0, 0)),
                  pl.BlockSpec((1, 512), lambda i: (0, 0)),
                  pl.BlockSpec((1, 512), lambda i: (0, 0)),
                  pl.BlockSpec((256, 512), lambda i: (0, 0)),
                  pl.BlockSpec((1, 256), lambda i: (0, 0)),
                  pl.BlockSpec((16, 256), lambda i: (0, 0)),
                  pl.BlockSpec((1, 16), lambda i: (0, 0))],
        out_specs=pl.BlockSpec((G, 16), lambda i: (0, 0)),
        out_shape=jax.ShapeDtypeStruct((G, 16), jnp.float32),
        scratch_shapes=[pltpu.VMEM((G, 512), jnp.float32)],
    )(h3pre, batchf, st, g, be, Wf1, bf1, Wf2p, bf2p)


def kernel(x, edge_index, batch, W1l, b1, W1r, g1, be1, W2l, b2, W2r, g2, be2,
           W3l, b3, W3r, g3, be3, Wf1, bf1, Wf2, bf2):
    src = edge_index[0].reshape(NW, NCHUNK, K)
    dst = edge_index[1].reshape(NW, NCHUNK, K)
    z16 = jnp.zeros((SLC, 16), jnp.float32)
    z64 = jnp.zeros((SLC, 64), jnp.float32)
    onesK = jnp.ones((K, 16), jnp.float32)

    p1, r1 = _pre1(x, W1l, W1r)
    a10, a11, dg0, dg1 = _sc_agg(16, True)(p1, src, dst, z16, onesK)
    h1pre, st1 = _combine(a10, a11, dg0, dg1, r1, None, b1.reshape(1, 16))
    h1, r2 = _norm(h1pre, st1, g1.reshape(1, 16), be1.reshape(1, 16), W2r)

    a20, a21 = _sc_agg(16, False)(h1, src, dst, z16)
    h2pre, st2 = _combine(a20, a21, dg0, dg1, r2, W2l, b2.reshape(1, 64))
    h2 = _norm(h2pre, st2, g2.reshape(1, 64), be2.reshape(1, 64), None)

    a30, a31 = _sc_agg(64, False)(h2, src, dst, z64)
    h3pre, st3 = _combine3(a30, a31, dg0, dg1, h2, W3l, W3r,
                           b3.reshape(1, 512))

    Wf2p = jnp.zeros((16, 256), jnp.float32).at[:OUT].set(Wf2)
    bf2p = jnp.zeros((1, 16), jnp.float32).at[0, :OUT].set(bf2)
    out = _final(h3pre, batch.astype(jnp.float32).reshape(N, 1), st3,
                 g3.reshape(1, 512), be3.reshape(1, 512), Wf1,
                 bf1.reshape(1, 256), Wf2p, bf2p)
    return out[:, :OUT]


# speedup vs baseline: 13.9280x; 1.0271x over previous
import functools

import jax
import jax.numpy as jnp
from jax import lax
from jax.experimental import pallas as pl
from jax.experimental.pallas import tpu as pltpu
from jax.experimental.pallas import tpu_sc as plsc

N, E, F, G, OUT = 10000, 320000, 128, 64, 10
NC, NS = 2, 16
NW = NC * NS
EW = E // NW
K = 125
NCHUNK = EW // K
SLC = 632
EPS = 1e-5


def _sc_agg(d, with_deg):
    mesh = plsc.VectorSubcoreMesh(core_axis_name="c", subcore_axis_name="s")
    outs = [jax.ShapeDtypeStruct((N, d), jnp.float32),
            jax.ShapeDtypeStruct((N, d), jnp.float32)]
    scratch = [
        pltpu.VMEM((NCHUNK, K), jnp.int32),
        pltpu.VMEM((NCHUNK, K), jnp.int32),
        pltpu.VMEM((K, d), jnp.float32),
        pltpu.VMEM((K, d), jnp.float32),
        pltpu.VMEM_SHARED((N, d), jnp.float32),
        pltpu.SemaphoreType.DMA,
        pltpu.SemaphoreType.DMA,
    ]
    if with_deg:
        outs += [jax.ShapeDtypeStruct((N, 16), jnp.float32),
                 jax.ShapeDtypeStruct((N, 16), jnp.float32)]
        scratch += [
            pltpu.VMEM((K, 16), jnp.float32),
            pltpu.VMEM_SHARED((N, 16), jnp.float32),
        ]

    def body(*refs):
        if with_deg:
            (vh, sh, dh, zh, ones_h, out0, out1, dg0, dg1,
             srcv, dstv, rows0, rows1, acc, sem0, sem1, onesv, dacc) = refs
        else:
            (vh, sh, dh, zh, out0, out1,
             srcv, dstv, rows0, rows1, acc, sem0, sem1) = refs
        c = lax.axis_index("c")
        s = lax.axis_index("s")
        wid = s * NC + c
        off = pl.multiple_of(jnp.minimum(s * SLC, N - SLC), 8)
        sl = pl.ds(off, SLC)
        pltpu.sync_copy(zh, acc.at[sl])
        if with_deg:
            pltpu.sync_copy(zh, dacc.at[sl])
            pltpu.sync_copy(ones_h, onesv)
        pltpu.sync_copy(sh.at[wid], srcv)
        pltpu.sync_copy(dh.at[wid], dstv)
        pltpu.async_copy(vh.at[srcv.at[0]], rows0, sem0)
        plsc.subcore_barrier()

        def scat(rows, j):
            pltpu.sync_copy(rows, acc.at[dstv.at[j]], add=True)
            if with_deg:
                pltpu.sync_copy(onesv, dacc.at[dstv.at[j]], add=True)

        def pair(i, carry):
            j = i * 2
            pltpu.async_copy(vh.at[srcv.at[j + 1]], rows1, sem1)
            pltpu.make_async_copy(vh.at[srcv.at[j]], rows0, sem0).wait()
            scat(rows0, j)

            @pl.when(j + 2 < NCHUNK)
            def _():
                pltpu.async_copy(vh.at[srcv.at[j + 2]], rows0, sem0)

            pltpu.make_async_copy(vh.at[srcv.at[j + 1]], rows1, sem1).wait()
            scat(rows1, j + 1)
            return carry

        lax.fori_loop(0, NCHUNK // 2, pair, 0)
        plsc.subcore_barrier()

        @pl.when(c == 0)
        def _():
            pltpu.sync_copy(acc.at[sl], out0.at[sl])
            if with_deg:
                pltpu.sync_copy(dacc.at[sl], dg0.at[sl])

        @pl.when(c == 1)
        def _():
            pltpu.sync_copy(acc.at[sl], out1.at[sl])
            if with_deg:
                pltpu.sync_copy(dacc.at[sl], dg1.at[sl])

    return pl.kernel(
        body, out_type=outs, scratch_types=scratch, mesh=mesh,
        compiler_params=pltpu.CompilerParams(use_tc_tiling_on_sc=False))


def _dotT(a, w):
    return lax.dot_general(a, w, (((1,), (1,)), ((), ())),
                           preferred_element_type=jnp.float32)


def _pre1(x, W1l, W1r):
    R = 1000
    NB = N // R

    def body(x_ref, wl_ref, wr_ref, p_ref, r_ref):
        xb = x_ref[...]
        p_ref[...] = _dotT(xb, wl_ref[...])
        r_ref[...] = _dotT(xb, wr_ref[...])

    return pl.pallas_call(
        body,
        grid=(NB,),
        in_specs=[pl.BlockSpec((R, F), lambda i: (i, 0)),
                  pl.BlockSpec((16, F), lambda i: (0, 0)),
                  pl.BlockSpec((16, F), lambda i: (0, 0))],
        out_specs=[pl.BlockSpec((R, 16), lambda i: (i, 0)),
                   pl.BlockSpec((R, 16), lambda i: (i, 0))],
        out_shape=[jax.ShapeDtypeStruct((N, 16), jnp.float32),
                   jax.ShapeDtypeStruct((N, 16), jnp.float32)],
    )(x, W1l, W1r)


def _fused12(a0, a1, d0, d1, root, Wl, b, g, be, Wn):
    do = root.shape[1]
    da = a0.shape[1]
    R = 1000
    NB = N // R
    have_w = Wl is not None
    have_n = Wn is not None

    def body(*refs):
        it = iter(refs)
        a0_ref, a1_ref, d0_ref, d1_ref, r_ref = [next(it) for _ in range(5)]
        w_ref = next(it) if have_w else None
        b_ref, g_ref, be_ref = [next(it) for _ in range(3)]
        wn_ref = next(it) if have_n else None
        h_ref = next(it)
        pn_ref = next(it) if have_n else None
        pre_ref, st_ref = next(it), next(it)
        p = pl.program_id(0)
        i = pl.program_id(1)

        @pl.when((p == 0) & (i == 0))
        def _():
            st_ref[...] = jnp.zeros_like(st_ref)

        @pl.when(p == 0)
        def _():
            deg = jnp.maximum(d0_ref[:, :1] + d1_ref[:, :1], 1.0)
            mean = (a0_ref[...] + a1_ref[...]) / deg
            if have_w:
                mean = _dotT(mean, w_ref[...])
            h = mean + b_ref[...] + r_ref[...]
            pre_ref[pl.ds(i * R, R), :] = h
            st_ref[0:1, :] += jnp.sum(h, axis=0, keepdims=True)
            st_ref[1:2, :] += jnp.sum(h * h, axis=0, keepdims=True)

        @pl.when(p == 1)
        def _():
            mu = st_ref[0:1, :] / N
            var = st_ref[1:2, :] / N - mu * mu
            scale = g_ref[...] * lax.rsqrt(var + EPS)
            h = jnp.maximum(
                (pre_ref[pl.ds(i * R, R), :] - mu) * scale + be_ref[...], 0.0)
            h_ref[...] = h
            if have_n:
                pn_ref[...] = _dotT(h, wn_ref[...])

    ph0 = lambda p, i: ((1 - p) * i, 0)
    ph1 = lambda p, i: (p * i, 0)
    cst = lambda p, i: (0, 0)
    in_specs = [pl.BlockSpec((R, da), ph0),
                pl.BlockSpec((R, da), ph0),
                pl.BlockSpec((R, 16), ph0),
                pl.BlockSpec((R, 16), ph0),
                pl.BlockSpec((R, do), ph0)]
    args = [a0, a1, d0, d1, root]
    if have_w:
        in_specs.append(pl.BlockSpec(Wl.shape, cst))
        args.append(Wl)
    in_specs += [pl.BlockSpec((1, do), cst)] * 3
    args += [b, g, be]
    out_specs = [pl.BlockSpec((R, do), ph1)]
    out_shape = [jax.ShapeDtypeStruct((N, do), jnp.float32)]
    if have_n:
        dn = Wn.shape[0]
        in_specs.append(pl.BlockSpec(Wn.shape, cst))
        args.append(Wn)
        out_specs.append(pl.BlockSpec((R, dn), ph1))
        out_shape.append(jax.ShapeDtypeStruct((N, dn), jnp.float32))
    res = pl.pallas_call(
        body,
        grid=(2, NB),
        in_specs=in_specs,
        out_specs=out_specs,
        out_shape=out_shape,
        scratch_shapes=[pltpu.VMEM((N, do), jnp.float32),
                        pltpu.VMEM((8, do), jnp.float32)],
    )(*args)
    return res if have_n else res[0]


def _fused3(a0, a1, d0, d1, h2, W3l, W3r, b, g, be, batchf,
            Wf1, bf1, Wf2p, bf2p):
    R = 400
    NB = N // R

    def body(a0_ref, a1_ref, d0_ref, d1_ref, h2_ref, wl_ref, wr_ref, b_ref,
             g_ref, be_ref, bt_ref, w1_ref, b1_ref, w2_ref, b2_ref,
             o_ref, pre_ref, st_ref, acc):
        p = pl.program_id(0)
        i = pl.program_id(1)

        @pl.when((p == 0) & (i == 0))
        def _():
            st_ref[...] = jnp.zeros_like(st_ref)
            acc[...] = jnp.zeros_like(acc)

        @pl.when(p == 0)
        def _():
            deg = jnp.maximum(d0_ref[:, :1] + d1_ref[:, :1], 1.0)
            mean = (a0_ref[...] + a1_ref[...]) / deg
            h = (_dotT(mean, wl_ref[...]) + b_ref[...]
                 + _dotT(h2_ref[...], wr_ref[...]))
            pre_ref[pl.ds(i * R, R), :] = h
            st_ref[0:1, :] += jnp.sum(h, axis=0, keepdims=True)
            st_ref[1:2, :] += jnp.sum(h * h, axis=0, keepdims=True)

        @pl.when(p == 1)
        def _():
            mu = st_ref[0:1, :] / N
            var = st_ref[1:2, :] / N - mu * mu
            scale = g_ref[...] * lax.rsqrt(var + EPS)
            h = jnp.maximum(
                (pre_ref[pl.ds(i * R, R), :] - mu) * scale + be_ref[...], 0.0)
            bb = bt_ref[...]
            cmin = jnp.min(bb).astype(jnp.int32)
            cmax = jnp.max(bb).astype(jnp.int32)

            def upd(c, carry):
                m = (bb == c.astype(jnp.float32))
                contrib = jnp.max(jnp.where(m, h, 0.0), axis=0, keepdims=True)
                row = pl.ds(c, 1)
                acc[row, :] = jnp.maximum(acc[row, :], contrib)
                return carry

            lax.fori_loop(cmin, cmax + 1, upd, 0)

            @pl.when(i == NB - 1)
            def _():
                pooled = acc[...]
                t = jnp.maximum(_dotT(pooled, w1_ref[...]) + b1_ref[...], 0.0)
                o_ref[...] = _dotT(t, w2_ref[...]) + b2_ref[...]

    ph0 = lambda p, i: ((1 - p) * i, 0)
    ph1 = lambda p, i: (p * i, 0)
    cst = lambda p, i: (0, 0)
    return pl.pallas_call(
        body,
        grid=(2, NB),
        in_specs=[pl.BlockSpec((R, 64), ph0),
                  pl.BlockSpec((R, 64), ph0),
                  pl.BlockSpec((R, 16), ph0),
                  pl.BlockSpec((R, 16), ph0),
                  pl.BlockSpec((R, 64), ph0),
                  pl.BlockSpec((512, 64), cst),
                  pl.BlockSpec((512, 64), cst),
                  pl.BlockSpec((1, 512), cst),
                  pl.BlockSpec((1, 512), cst),
                  pl.BlockSpec((1, 512), cst),
                  pl.BlockSpec((R, 1), ph1),
                  pl.BlockSpec((256, 512), cst),
                  pl.BlockSpec((1, 256), cst),
                  pl.BlockSpec((16, 256), cst),
                  pl.BlockSpec((1, 16), cst)],
        out_specs=pl.BlockSpec((G, 16), cst),
        out_shape=jax.ShapeDtypeStruct((G, 16), jnp.float32),
        scratch_shapes=[pltpu.VMEM((N, 512), jnp.float32),
                        pltpu.VMEM((8, 512), jnp.float32),
                        pltpu.VMEM((G, 512), jnp.float32)],
    )(a0, a1, d0, d1, h2, W3l, W3r, b, g, be, batchf, Wf1, bf1, Wf2p, bf2p)


def kernel(x, edge_index, batch, W1l, b1, W1r, g1, be1, W2l, b2, W2r, g2, be2,
           W3l, b3, W3r, g3, be3, Wf1, bf1, Wf2, bf2):
    src = edge_index[0].reshape(NW, NCHUNK, K)
    dst = edge_index[1].reshape(NW, NCHUNK, K)
    z16 = jnp.zeros((SLC, 16), jnp.float32)
    z64 = jnp.zeros((SLC, 64), jnp.float32)
    onesK = jnp.ones((K, 16), jnp.float32)

    p1, r1 = _pre1(x, W1l, W1r)
    a10, a11, dg0, dg1 = _sc_agg(16, True)(p1, src, dst, z16, onesK)
    h1, r2 = _fused12(a10, a11, dg0, dg1, r1, None, b1.reshape(1, 16),
                      g1.reshape(1, 16), be1.reshape(1, 16), W2r)

    a20, a21 = _sc_agg(16, False)(h1, src, dst, z16)
    h2 = _fused12(a20, a21, dg0, dg1, r2, W2l, b2.reshape(1, 64),
                  g2.reshape(1, 64), be2.reshape(1, 64), None)

    a30, a31 = _sc_agg(64, False)(h2, src, dst, z64)
    Wf2p = jnp.zeros((16, 256), jnp.float32).at[:OUT].set(Wf2)
    bf2p = jnp.zeros((1, 16), jnp.float32).at[0, :OUT].set(bf2)
    out = _fused3(a30, a31, dg0, dg1, h2, W3l, W3r, b3.reshape(1, 512),
                  g3.reshape(1, 512), be3.reshape(1, 512),
                  batch.astype(jnp.float32).reshape(N, 1), Wf1,
                  bf1.reshape(1, 256), Wf2p, bf2p)
    return out[:, :OUT]


# speedup vs baseline: 16.2390x; 1.1659x over previous
import functools

import jax
import jax.numpy as jnp
from jax import lax
from jax.experimental import pallas as pl
from jax.experimental.pallas import tpu as pltpu
from jax.experimental.pallas import tpu_sc as plsc

N, E, F, G, OUT = 10000, 320000, 128, 64, 10
NC, NS = 2, 16
NW = NC * NS
EW = E // NW
K = 125
NCHUNK = EW // K
DEPTH = 4
SLC = 632
EPS = 1e-5


def _sc_agg(d, with_deg):
    mesh = plsc.VectorSubcoreMesh(core_axis_name="c", subcore_axis_name="s")
    outs = [jax.ShapeDtypeStruct((N, d), jnp.float32),
            jax.ShapeDtypeStruct((N, d), jnp.float32)]
    scratch = [
        pltpu.VMEM((NCHUNK, K), jnp.int32),
        pltpu.VMEM((NCHUNK, K), jnp.int32),
        [pltpu.VMEM((K, d), jnp.float32) for _ in range(DEPTH)],
        pltpu.VMEM_SHARED((N, d), jnp.float32),
        [pltpu.SemaphoreType.DMA for _ in range(DEPTH)],
        [pltpu.SemaphoreType.DMA for _ in range(DEPTH)],
    ]
    if with_deg:
        outs += [jax.ShapeDtypeStruct((N, 16), jnp.float32),
                 jax.ShapeDtypeStruct((N, 16), jnp.float32)]
        scratch += [
            pltpu.VMEM((K, 16), jnp.float32),
            pltpu.VMEM_SHARED((N, 16), jnp.float32),
            pltpu.SemaphoreType.DMA,
        ]

    def body(*refs):
        if with_deg:
            (vh, sh, dh, zh, ones_h, out0, out1, dg0, dg1,
             srcv, dstv, rows, acc, gsem, ssem, onesv, dacc, dsem) = refs
        else:
            (vh, sh, dh, zh, out0, out1,
             srcv, dstv, rows, acc, gsem, ssem) = refs
        c = lax.axis_index("c")
        s = lax.axis_index("s")
        wid = s * NC + c
        off = pl.multiple_of(jnp.minimum(s * SLC, N - SLC), 8)
        sl = pl.ds(off, SLC)
        pltpu.sync_copy(zh, acc.at[sl])
        if with_deg:
            pltpu.sync_copy(zh, dacc.at[sl])
            pltpu.sync_copy(ones_h, onesv)
        pltpu.sync_copy(sh.at[wid], srcv)
        pltpu.sync_copy(dh.at[wid], dstv)
        for t in range(DEPTH):
            pltpu.async_copy(vh.at[srcv.at[t]], rows[t], gsem[t])
        plsc.subcore_barrier()

        def step(i, carry):
            for t in range(DEPTH):
                j = i * DEPTH + t
                pltpu.make_async_copy(vh.at[srcv.at[j]], rows[t],
                                      gsem[t]).wait()
                pltpu.async_copy(rows[t], acc.at[dstv.at[j]], ssem[t],
                                 add=True)
                if with_deg:
                    pltpu.async_copy(onesv, dacc.at[dstv.at[j]], dsem,
                                     add=True)
            for t in range(DEPTH):
                j = i * DEPTH + t

                @pl.when(j + DEPTH < NCHUNK)
                def _():
                    pltpu.make_async_copy(rows[t], acc.at[dstv.at[j]],
                                          ssem[t]).wait()
                    pltpu.async_copy(vh.at[srcv.at[j + DEPTH]], rows[t],
                                     gsem[t])
            return carry

        lax.fori_loop(0, NCHUNK // DEPTH, step, 0)
        for t in range(DEPTH):
            pltpu.make_async_copy(rows[t], acc.at[dstv.at[NCHUNK - DEPTH + t]],
                                  ssem[t]).wait()

        if with_deg:
            def drain(j, carry):
                pltpu.make_async_copy(onesv, dacc.at[dstv.at[j]], dsem).wait()
                return carry
            lax.fori_loop(0, NCHUNK, drain, 0)
        plsc.subcore_barrier()

        @pl.when(c == 0)
        def _():
            pltpu.sync_copy(acc.at[sl], out0.at[sl])
            if with_deg:
                pltpu.sync_copy(dacc.at[sl], dg0.at[sl])

        @pl.when(c == 1)
        def _():
            pltpu.sync_copy(acc.at[sl], out1.at[sl])
            if with_deg:
                pltpu.sync_copy(dacc.at[sl], dg1.at[sl])

    return pl.kernel(
        body, out_type=outs, scratch_types=scratch, mesh=mesh,
        compiler_params=pltpu.CompilerParams(use_tc_tiling_on_sc=False))


def _dotT(a, w):
    return lax.dot_general(a, w, (((1,), (1,)), ((), ())),
                           preferred_element_type=jnp.float32)


def _pre1(x, W1l, W1r):
    R = 2000
    NB = N // R

    def body(x_ref, wl_ref, wr_ref, p_ref, r_ref):
        xb = x_ref[...]
        p_ref[...] = _dotT(xb, wl_ref[...])
        r_ref[...] = _dotT(xb, wr_ref[...])

    return pl.pallas_call(
        body,
        grid=(NB,),
        in_specs=[pl.BlockSpec((R, F), lambda i: (i, 0)),
                  pl.BlockSpec((16, F), lambda i: (0, 0)),
                  pl.BlockSpec((16, F), lambda i: (0, 0))],
        out_specs=[pl.BlockSpec((R, 16), lambda i: (i, 0)),
                   pl.BlockSpec((R, 16), lambda i: (i, 0))],
        out_shape=[jax.ShapeDtypeStruct((N, 16), jnp.float32),
                   jax.ShapeDtypeStruct((N, 16), jnp.float32)],
    )(x, W1l, W1r)


def _fused12(a0, a1, d0, d1, root, Wl, b, g, be, Wn):
    do = root.shape[1]
    da = a0.shape[1]
    R = 2000
    NB = N // R
    have_w = Wl is not None
    have_n = Wn is not None

    def body(*refs):
        it = iter(refs)
        a0_ref, a1_ref, d0_ref, d1_ref, r_ref = [next(it) for _ in range(5)]
        w_ref = next(it) if have_w else None
        b_ref, g_ref, be_ref = [next(it) for _ in range(3)]
        wn_ref = next(it) if have_n else None
        h_ref = next(it)
        pn_ref = next(it) if have_n else None
        pre_ref, st_ref = next(it), next(it)
        p = pl.program_id(0)
        i = pl.program_id(1)

        @pl.when((p == 0) & (i == 0))
        def _():
            st_ref[...] = jnp.zeros_like(st_ref)

        @pl.when(p == 0)
        def _():
            deg = jnp.maximum(d0_ref[:, :1] + d1_ref[:, :1], 1.0)
            mean = (a0_ref[...] + a1_ref[...]) / deg
            if have_w:
                mean = _dotT(mean, w_ref[...])
            h = mean + b_ref[...] + r_ref[...]
            pre_ref[pl.ds(i * R, R), :] = h
            st_ref[0:1, :] += jnp.sum(h, axis=0, keepdims=True)
            st_ref[1:2, :] += jnp.sum(h * h, axis=0, keepdims=True)

        @pl.when(p == 1)
        def _():
            mu = st_ref[0:1, :] / N
            var = st_ref[1:2, :] / N - mu * mu
            scale = g_ref[...] * lax.rsqrt(var + EPS)
            h = jnp.maximum(
                (pre_ref[pl.ds(i * R, R), :] - mu) * scale + be_ref[...], 0.0)
            h_ref[...] = h
            if have_n:
                pn_ref[...] = _dotT(h, wn_ref[...])

    ph0 = lambda p, i: ((1 - p) * i, 0)
    ph1 = lambda p, i: (p * i, 0)
    cst = lambda p, i: (0, 0)
    in_specs = [pl.BlockSpec((R, da), ph0),
                pl.BlockSpec((R, da), ph0),
                pl.BlockSpec((R, 16), ph0),
                pl.BlockSpec((R, 16), ph0),
                pl.BlockSpec((R, do), ph0)]
    args = [a0, a1, d0, d1, root]
    if have_w:
        in_specs.append(pl.BlockSpec(Wl.shape, cst))
        args.append(Wl)
    in_specs += [pl.BlockSpec((1, do), cst)] * 3
    args += [b, g, be]
    out_specs = [pl.BlockSpec((R, do), ph1)]
    out_shape = [jax.ShapeDtypeStruct((N, do), jnp.float32)]
    if have_n:
        dn = Wn.shape[0]
        in_specs.append(pl.BlockSpec(Wn.shape, cst))
        args.append(Wn)
        out_specs.append(pl.BlockSpec((R, dn), ph1))
        out_shape.append(jax.ShapeDtypeStruct((N, dn), jnp.float32))
    res = pl.pallas_call(
        body,
        grid=(2, NB),
        in_specs=in_specs,
        out_specs=out_specs,
        out_shape=out_shape,
        scratch_shapes=[pltpu.VMEM((N, do), jnp.float32),
                        pltpu.VMEM((8, do), jnp.float32)],
    )(*args)
    return res if have_n else res[0]


def _fused3(a0, a1, d0, d1, h2, W3l, W3r, b, g, be, batchf,
            Wf1, bf1, Wf2p, bf2p):
    R = 400
    NB = N // R

    def body(a0_ref, a1_ref, d0_ref, d1_ref, h2_ref, wl_ref, wr_ref, b_ref,
             g_ref, be_ref, bt_ref, w1_ref, b1_ref, w2_ref, b2_ref,
             o_ref, pre_ref, st_ref, acc):
        p = pl.program_id(0)
        i = pl.program_id(1)

        @pl.when((p == 0) & (i == 0))
        def _():
            st_ref[...] = jnp.zeros_like(st_ref)
            acc[...] = jnp.zeros_like(acc)

        @pl.when(p == 0)
        def _():
            deg = jnp.maximum(d0_ref[:, :1] + d1_ref[:, :1], 1.0)
            mean = (a0_ref[...] + a1_ref[...]) / deg
            h = (_dotT(mean, wl_ref[...]) + b_ref[...]
                 + _dotT(h2_ref[...], wr_ref[...]))
            pre_ref[pl.ds(i * R, R), :] = h
            st_ref[0:1, :] += jnp.sum(h, axis=0, keepdims=True)
            st_ref[1:2, :] += jnp.sum(h * h, axis=0, keepdims=True)

        @pl.when(p == 1)
        def _():
            mu = st_ref[0:1, :] / N
            var = st_ref[1:2, :] / N - mu * mu
            scale = g_ref[...] * lax.rsqrt(var + EPS)
            h = jnp.maximum(
                (pre_ref[pl.ds(i * R, R), :] - mu) * scale + be_ref[...], 0.0)
            bb = bt_ref[...]
            cmin = jnp.min(bb).astype(jnp.int32)
            cmax = jnp.max(bb).astype(jnp.int32)

            def upd(c, carry):
                m = (bb == c.astype(jnp.float32))
                contrib = jnp.max(jnp.where(m, h, 0.0), axis=0, keepdims=True)
                row = pl.ds(c, 1)
                acc[row, :] = jnp.maximum(acc[row, :], contrib)
                return carry

            lax.fori_loop(cmin, cmax + 1, upd, 0)

            @pl.when(i == NB - 1)
            def _():
                pooled = acc[...]
                t = jnp.maximum(_dotT(pooled, w1_ref[...]) + b1_ref[...], 0.0)
                o_ref[...] = _dotT(t, w2_ref[...]) + b2_ref[...]

    ph0 = lambda p, i: ((1 - p) * i, 0)
    ph1 = lambda p, i: (p * i, 0)
    cst = lambda p, i: (0, 0)
    return pl.pallas_call(
        body,
        grid=(2, NB),
        in_specs=[pl.BlockSpec((R, 64), ph0),
                  pl.BlockSpec((R, 64), ph0),
                  pl.BlockSpec((R, 16), ph0),
                  pl.BlockSpec((R, 16), ph0),
                  pl.BlockSpec((R, 64), ph0),
                  pl.BlockSpec((512, 64), cst),
                  pl.BlockSpec((512, 64), cst),
                  pl.BlockSpec((1, 512), cst),
                  pl.BlockSpec((1, 512), cst),
                  pl.BlockSpec((1, 512), cst),
                  pl.BlockSpec((R, 1), ph1),
                  pl.BlockSpec((256, 512), cst),
                  pl.BlockSpec((1, 256), cst),
                  pl.BlockSpec((16, 256), cst),
                  pl.BlockSpec((1, 16), cst)],
        out_specs=pl.BlockSpec((G, 16), cst),
        out_shape=jax.ShapeDtypeStruct((G, 16), jnp.float32),
        scratch_shapes=[pltpu.VMEM((N, 512), jnp.float32),
                        pltpu.VMEM((8, 512), jnp.float32),
                        pltpu.VMEM((G, 512), jnp.float32)],
    )(a0, a1, d0, d1, h2, W3l, W3r, b, g, be, batchf, Wf1, bf1, Wf2p, bf2p)


def kernel(x, edge_index, batch, W1l, b1, W1r, g1, be1, W2l, b2, W2r, g2, be2,
           W3l, b3, W3r, g3, be3, Wf1, bf1, Wf2, bf2):
    src = edge_index[0].reshape(NW, NCHUNK, K)
    dst = edge_index[1].reshape(NW, NCHUNK, K)
    z16 = jnp.zeros((SLC, 16), jnp.float32)
    z64 = jnp.zeros((SLC, 64), jnp.float32)
    onesK = jnp.ones((K, 16), jnp.float32)

    p1, r1 = _pre1(x, W1l, W1r)
    a10, a11, dg0, dg1 = _sc_agg(16, True)(p1, src, dst, z16, onesK)
    h1, r2 = _fused12(a10, a11, dg0, dg1, r1, None, b1.reshape(1, 16),
                      g1.reshape(1, 16), be1.reshape(1, 16), W2r)

    a20, a21 = _sc_agg(16, False)(h1, src, dst, z16)
    h2 = _fused12(a20, a21, dg0, dg1, r2, W2l, b2.reshape(1, 64),
                  g2.reshape(1, 64), be2.reshape(1, 64), None)

    a30, a31 = _sc_agg(64, False)(h2, src, dst, z64)
    Wf2p = jnp.zeros((16, 256), jnp.float32).at[:OUT].set(Wf2)
    bf2p = jnp.zeros((1, 16), jnp.float32).at[0, :OUT].set(bf2)
    out = _fused3(a30, a31, dg0, dg1, h2, W3l, W3r, b3.reshape(1, 512),
                  g3.reshape(1, 512), be3.reshape(1, 512),
                  batch.astype(jnp.float32).reshape(N, 1), Wf1,
                  bf1.reshape(1, 256), Wf2p, bf2p)
    return out[:, :OUT]
